# baseline (speedup 1.0000x reference)
import functools

import jax
import jax.numpy as jnp
from jax import lax
from jax.experimental import pallas as pl
from jax.experimental.pallas import tpu as pltpu
from jax.experimental.pallas import tpu_sc as plsc

_BATCH = 4096
_SEQ = 50
_D = 64
_B = _BATCH * _SEQ
_NW = 32
_BPW = _B // _NW
_NB = _BATCH // _NW
_CBATCH = 4
_CH = _CBATCH * _SEQ
_NCH = _NB // _CBATCH
_NSEM = 4


def _build():
  mesh = plsc.VectorSubcoreMesh(core_axis_name="c", subcore_axis_name="s")

  @functools.partial(
      pl.kernel,
      mesh=mesh,
      out_type=jax.ShapeDtypeStruct((_BATCH, _SEQ, _D), jnp.float32),
      scratch_types=[
          pltpu.VMEM((_BPW,), jnp.int32),
          pltpu.VMEM((2, _CBATCH, _SEQ, _D), jnp.float32),
          pltpu.SemaphoreType.DMA,
          [pltpu.SemaphoreType.DMA] * (2 * _NSEM),
          [pltpu.SemaphoreType.DMA] * 2,
      ],
  )
  def k(idx_hbm, table_hbm, out_hbm, idx_v, bufs, isem, gsems, wsem):
    wid = lax.axis_index("s") * 2 + lax.axis_index("c")
    pltpu.async_copy(idx_hbm.at[wid], idx_v, isem).wait()

    def out_block(j):
      return out_hbm.at[pl.ds(wid * _NB + j * _CBATCH, _CBATCH)]

    def issue(j, b):
      buf = bufs.at[b]

      def do_row(i, sem_i, r):
        pltpu.async_copy(
            table_hbm.at[pl.ds(r, 1)],
            buf.at[i // _SEQ, pl.ds(i % _SEQ, 1)],
            gsems[b * _NSEM + sem_i],
        )

      def grp_body(g, _):
        vec = idx_v[pl.ds(j * _CH + g * 16, 16)]
        for lane in range(16):
          do_row(g * 16 + lane, lane % _NSEM, vec[lane])
        return 0

      lax.fori_loop(0, _CH // 16, grp_body, 0)
      vec = idx_v[pl.ds(j * _CH + (_CH // 16) * 16, 16)]
      for lane in range(_CH - (_CH // 16) * 16):
        do_row((_CH // 16) * 16 + lane, lane % _NSEM, vec[lane])

    _drain_rows = [len([i for i in range(_CH) if i % _NSEM == s])
                   for s in range(_NSEM)]

    def drain_and_flush(j, b):
      buf = bufs.at[b]
      dst = out_block(j)
      assert all(n == _SEQ for n in _drain_rows)
      for s in range(_NSEM):
        pltpu.make_async_copy(
            out_hbm.at[0],
            bufs.at[b, 0],
            gsems[b * _NSEM + s],
        ).wait()
      pltpu.async_copy(buf, dst, wsem[b])

    issue(0, 0)
    issue(1, 1)

    def loop_body(j2, _):
      for b in range(2):
        j = j2 * 2 + b
        drain_and_flush(j, b)

        @pl.when(j + 2 < _NCH)
        def _():
          pltpu.make_async_copy(out_block(j), bufs.at[b], wsem[b]).wait()
          issue(j + 2, b)

      return 0

    lax.fori_loop(0, _NCH // 2, loop_body, 0)
    pltpu.make_async_copy(out_block(_NCH - 2), bufs.at[0], wsem[0]).wait()
    pltpu.make_async_copy(out_block(_NCH - 1), bufs.at[1], wsem[1]).wait()

  return k


_gather_kernel = _build()


def kernel(tok_idxs, embed):
  idx = tok_idxs.reshape(_NW, _BPW)
  return _gather_kernel(idx, embed)
